# Initial kernel scaffold; baseline (speedup 1.0000x reference)
import jax
import jax.numpy as jnp
from jax.experimental import pallas as pl


def kernel(x, table):
    raise NotImplementedError("write your pallas kernel here")


# speedup vs baseline: 1.3055x; 1.3055x over previous
import functools
import math

import jax
import jax.numpy as jnp
from jax import lax
from jax.experimental import pallas as pl
from jax.experimental.pallas import tpu as pltpu
from jax.experimental.pallas import tpu_sc as plsc

D_MODEL = 1024
SCALE = math.sqrt(D_MODEL)
LANES = 16
VREGS_PER_ROW = D_MODEL // LANES
NUM_CORES = 2
NUM_SUBCORES = 16
NUM_WORKERS = NUM_CORES * NUM_SUBCORES
CHUNK = 32


def _sc_body(n_chunks, table_hbm, idx_hbm, out_hbm, idx_v, buf0, buf1,
             sem0, sem1):
    wid = lax.axis_index("s") * NUM_CORES + lax.axis_index("c")
    base = wid * (n_chunks * CHUNK)
    pltpu.sync_copy(idx_hbm.at[wid], idx_v)

    bufs = (buf0, buf1)
    sems = (sem0, sem1)
    handles = {}
    handles[0] = pltpu.async_copy(table_hbm.at[idx_v.at[0]], buf0, sem0)
    for j in range(n_chunks):
        cur = bufs[j % 2]
        if j + 1 < n_chunks:
            handles[(j + 1) % 2] = pltpu.async_copy(
                table_hbm.at[idx_v.at[j + 1]], bufs[(j + 1) % 2],
                sems[(j + 1) % 2])
        handles[j % 2].wait()

        def scale_row(r, carry):
            for v in range(VREGS_PER_ROW):
                sl = pl.ds(v * LANES, LANES)
                cur[r, sl] = cur[r, sl] * SCALE
            return carry

        lax.fori_loop(0, CHUNK, scale_row, 0, unroll=False)
        pltpu.sync_copy(cur, out_hbm.at[pl.ds(base + j * CHUNK, CHUNK)])


@functools.lru_cache(maxsize=None)
def _make_lookup(batch):
    assert batch % (NUM_WORKERS * CHUNK) == 0
    n_chunks = batch // (NUM_WORKERS * CHUNK)
    mesh = plsc.VectorSubcoreMesh(core_axis_name="c", subcore_axis_name="s")
    return pl.kernel(
        functools.partial(_sc_body, n_chunks),
        out_type=jax.ShapeDtypeStruct((batch, D_MODEL), jnp.float32),
        mesh=mesh,
        scratch_types=[
            pltpu.VMEM((n_chunks, CHUNK), jnp.int32),
            pltpu.VMEM((CHUNK, D_MODEL), jnp.float32),
            pltpu.VMEM((CHUNK, D_MODEL), jnp.float32),
            pltpu.SemaphoreType.DMA,
            pltpu.SemaphoreType.DMA,
        ],
    )


def kernel(x, table):
    batch = x.size
    n_chunks = batch // (NUM_WORKERS * CHUNK)
    idx = x.astype(jnp.int32).reshape(NUM_WORKERS, n_chunks, CHUNK)
    out = _make_lookup(batch)(table, idx)
    return out.reshape(*x.shape, D_MODEL)


# speedup vs baseline: 1.3815x; 1.0582x over previous
import functools
import math

import jax
import jax.numpy as jnp
from jax import lax
from jax.experimental import pallas as pl
from jax.experimental.pallas import tpu as pltpu
from jax.experimental.pallas import tpu_sc as plsc

D_MODEL = 1024
SCALE = math.sqrt(D_MODEL)
LANES = 16
VREGS_PER_ROW = D_MODEL // LANES
NUM_CORES = 2
NUM_SUBCORES = 16
NUM_WORKERS = NUM_CORES * NUM_SUBCORES
CHUNK = 32


def _sc_body(n_chunks, table_hbm, idx_hbm, out_hbm, idx_v, buf0, buf1,
             gsem0, gsem1, ssem0, ssem1):
    wid = lax.axis_index("s") * NUM_CORES + lax.axis_index("c")
    base = wid * (n_chunks * CHUNK)
    pltpu.sync_copy(idx_hbm.at[wid], idx_v)

    bufs = (buf0, buf1)
    gsems = (gsem0, gsem1)
    ssems = (ssem0, ssem1)
    gather = {}
    store = {}
    gather[0] = pltpu.async_copy(table_hbm.at[idx_v.at[0]], buf0, gsem0)
    for j in range(n_chunks):
        b = j % 2
        if j + 1 < n_chunks:
            nb = (j + 1) % 2
            if j >= 1:
                store[nb].wait()
            gather[nb] = pltpu.async_copy(
                table_hbm.at[idx_v.at[j + 1]], bufs[nb], gsems[nb])
        gather[b].wait()
        cur = bufs[b]

        def scale_row(r, carry):
            for v in range(VREGS_PER_ROW):
                sl = pl.ds(v * LANES, LANES)
                cur[r, sl] = cur[r, sl] * SCALE
            return carry

        lax.fori_loop(0, CHUNK, scale_row, 0, unroll=False)
        store[b] = pltpu.async_copy(
            cur, out_hbm.at[pl.ds(base + j * CHUNK, CHUNK)], ssems[b])
    store[(n_chunks - 2) % 2].wait()
    store[(n_chunks - 1) % 2].wait()


@functools.lru_cache(maxsize=None)
def _make_lookup(batch):
    assert batch % (NUM_WORKERS * CHUNK) == 0
    n_chunks = batch // (NUM_WORKERS * CHUNK)
    mesh = plsc.VectorSubcoreMesh(core_axis_name="c", subcore_axis_name="s")
    return pl.kernel(
        functools.partial(_sc_body, n_chunks),
        out_type=jax.ShapeDtypeStruct((batch, D_MODEL), jnp.float32),
        mesh=mesh,
        scratch_types=[
            pltpu.VMEM((n_chunks, CHUNK), jnp.int32),
            pltpu.VMEM((CHUNK, D_MODEL), jnp.float32),
            pltpu.VMEM((CHUNK, D_MODEL), jnp.float32),
            pltpu.SemaphoreType.DMA,
            pltpu.SemaphoreType.DMA,
            pltpu.SemaphoreType.DMA,
            pltpu.SemaphoreType.DMA,
        ],
    )


def kernel(x, table):
    batch = x.size
    n_chunks = batch // (NUM_WORKERS * CHUNK)
    idx = x.astype(jnp.int32).reshape(NUM_WORKERS, n_chunks, CHUNK)
    out = _make_lookup(batch)(table, idx)
    return out.reshape(*x.shape, D_MODEL)
